# Initial kernel scaffold; baseline (speedup 1.0000x reference)
import jax
import jax.numpy as jnp
from jax.experimental import pallas as pl


def kernel(x, edge_index, batch, params):
    raise NotImplementedError("write your pallas kernel here")


# speedup vs baseline: 2.8180x; 2.8180x over previous
import functools

import jax
import jax.numpy as jnp
from jax import lax
from jax.experimental import pallas as pl
from jax.experimental.pallas import tpu as pltpu
from jax.experimental.pallas import tpu_sc as plsc

N_NODES = 10000
N_EDGES = 320000
F = 128
NUM_GRAPHS = 16
MSG_EPS = 1e-7
LN_EPS = 1e-5

NC = 2
NS = 16
EG = 64

E_ROWS = 5120
E_PAD = E_ROWS * EG - N_EDGES
ROWS_PER_TILE = E_ROWS // NS
W = 5120
S = 4992
N_PAD = S + W
TRASH_A = S
TRASH_B = N_NODES - S

BLK = 2000
GRID = N_NODES // BLK


ACC_ROWS_PER_TILE = W // NS


def _edge_body(p_hbm, q_hbm, src_hbm, dst2_hbm, zeros_hbm,
               den_hbm, num_hbm,
               src_v, dst_v, rows_v, sem, acc):
    c = lax.axis_index("c")
    s = lax.axis_index("s")
    a0 = s * ACC_ROWS_PER_TILE
    asl = pl.ds(a0, ACC_ROWS_PER_TILE)
    r0 = s * ROWS_PER_TILE

    pltpu.sync_copy(src_hbm.at[pl.ds(r0, ROWS_PER_TILE)], src_v)

    def one_pass(p, carry):
        pltpu.sync_copy(zeros_hbm, acc.at[asl])
        pltpu.sync_copy(dst2_hbm.at[p, pl.ds(r0, ROWS_PER_TILE)], dst_v)
        plsc.subcore_barrier()

        def body(j, carry2):
            @pl.when(c == 0)
            def _():
                pltpu.async_copy(p_hbm.at[src_v.at[j]], rows_v, sem).wait()

            @pl.when(c == 1)
            def _():
                pltpu.async_copy(q_hbm.at[src_v.at[j]], rows_v, sem).wait()

            pltpu.sync_copy(rows_v, acc.at[dst_v.at[j]], add=True)
            return carry2

        lax.fori_loop(0, ROWS_PER_TILE, body, 0)
        plsc.subcore_barrier()

        osl = pl.ds(p * S + a0, ACC_ROWS_PER_TILE)

        @pl.when(c == 0)
        def _():
            pltpu.sync_copy(acc.at[asl], den_hbm.at[osl])

        @pl.when(c == 1)
        def _():
            pltpu.sync_copy(acc.at[asl], num_hbm.at[osl])

        return carry

    lax.fori_loop(0, 2, one_pass, 0)


def _edge_pass(p, q, srcp, dst2):
    zeros_blk = jnp.zeros((ACC_ROWS_PER_TILE, F), jnp.float32)
    mesh = plsc.VectorSubcoreMesh(core_axis_name="c", subcore_axis_name="s",
                                  num_cores=NC, num_subcores=NS)
    f = pl.kernel(
        _edge_body,
        out_type=(jax.ShapeDtypeStruct((N_PAD, F), jnp.float32),
                  jax.ShapeDtypeStruct((N_PAD, F), jnp.float32)),
        mesh=mesh,
        scratch_types=[
            pltpu.VMEM((ROWS_PER_TILE, EG), jnp.int32),
            pltpu.VMEM((ROWS_PER_TILE, EG), jnp.int32),
            pltpu.VMEM((EG, F), jnp.float32),
            pltpu.SemaphoreType.DMA,
            pltpu.VMEM_SHARED((W, F), jnp.float32),
        ],
    )
    return f(p, q, srcp, dst2, zeros_blk)


def _ln(v, g, b):
    mu = jnp.mean(v, axis=-1, keepdims=True)
    var = jnp.mean((v - mu) ** 2, axis=-1, keepdims=True)
    return (v - mu) * lax.rsqrt(var + LN_EPS) * g + b


def _pre(h, ln_g, ln_b, t):
    z = jax.nn.relu(_ln(h, ln_g, ln_b))
    g = z + MSG_EPS
    p = jnp.exp(t * g)
    return z, p, g * p


def _mlp_post(h, z, den, num, w1, b1, mg, mb, w2, b2):
    aggr = num / (den + 1e-16)
    out = aggr + z
    u = jnp.dot(out, w1, preferred_element_type=jnp.float32) + b1
    u = jax.nn.relu(_ln(u, mg, mb))
    v = jnp.dot(u, w2, preferred_element_type=jnp.float32) + b2
    return h + v


def _enc_pre_body(x_ref, w_ref, b_ref, g_ref, bb_ref, t_ref,
                  h_ref, z_ref, p_ref, q_ref):
    h = jnp.dot(x_ref[...], w_ref[...],
                preferred_element_type=jnp.float32) + b_ref[0:1, :]
    z, p, q = _pre(h, g_ref[0:1, :], bb_ref[0:1, :], t_ref[0:1, :])
    h_ref[...] = h
    z_ref[...] = z
    p_ref[...] = p
    q_ref[...] = q


def _post_pre_body(h_ref, z_ref, den_ref, num_ref,
                   w1_ref, b1_ref, mg_ref, mb_ref, w2_ref, b2_ref,
                   g2_ref, bb2_ref, t2_ref,
                   h2_ref, z2_ref, p_ref, q_ref):
    hn = _mlp_post(h_ref[...], z_ref[...], den_ref[...], num_ref[...],
                   w1_ref[...], b1_ref[0:1, :], mg_ref[0:1, :], mb_ref[0:1, :],
                   w2_ref[...], b2_ref[0:1, :])
    z2, p, q = _pre(hn, g2_ref[0:1, :], bb2_ref[0:1, :], t2_ref[0:1, :])
    h2_ref[...] = hn
    z2_ref[...] = z2
    p_ref[...] = p
    q_ref[...] = q


def _post_pool_body(h_ref, z_ref, den_ref, num_ref,
                    w1_ref, b1_ref, mg_ref, mb_ref, w2_ref, b2_ref,
                    bf_ref, cw_ref, cb_ref,
                    out_ref, acc):
    i = pl.program_id(0)
    hn = _mlp_post(h_ref[...], z_ref[...], den_ref[...], num_ref[...],
                   w1_ref[...], b1_ref[0:1, :], mg_ref[0:1, :], mb_ref[0:1, :],
                   w2_ref[...], b2_ref[0:1, :])
    bvals = bf_ref[:, 0:NUM_GRAPHS]
    gid = lax.broadcasted_iota(jnp.int32, (BLK, NUM_GRAPHS), 1)
    oh = jnp.where(bvals == gid.astype(jnp.float32), 1.0, 0.0)
    part = lax.dot_general(oh, hn, (((0,), (0,)), ((), ())),
                           preferred_element_type=jnp.float32)

    @pl.when(i == 0)
    def _():
        acc[...] = part

    @pl.when(i > 0)
    def _():
        acc[...] = acc[...] + part

    out_ref[...] = jnp.dot(acc[...], cw_ref[...],
                           preferred_element_type=jnp.float32) + cb_ref[0:1, :]


def _full(shape):
    return pl.BlockSpec(shape, lambda i: (0, 0))


def _rows(width):
    return pl.BlockSpec((BLK, width), lambda i: (i, 0))


_N_SHAPE = jax.ShapeDtypeStruct((N_NODES, F), jnp.float32)


def _enc_pre(x, w, b, g, bb, t):
    return pl.pallas_call(
        _enc_pre_body,
        grid=(GRID,),
        in_specs=[_rows(F), _full((F, F)), _full((8, F)), _full((8, F)),
                  _full((8, F)), _full((8, F))],
        out_specs=[_rows(F)] * 4,
        out_shape=[_N_SHAPE] * 4,
    )(x, w, b, g, bb, t)


def _post_pre(h, z, den, num, w1, b1, mg, mb, w2, b2, g2, bb2, t2):
    return pl.pallas_call(
        _post_pre_body,
        grid=(GRID,),
        in_specs=[_rows(F)] * 4 + [
            _full((F, 2 * F)), _full((8, 2 * F)), _full((8, 2 * F)),
            _full((8, 2 * F)), _full((2 * F, F)), _full((8, F)),
            _full((8, F)), _full((8, F)), _full((8, F))],
        out_specs=[_rows(F)] * 4,
        out_shape=[_N_SHAPE] * 4,
    )(h, z, den, num, w1, b1, mg, mb, w2, b2, g2, bb2, t2)


def _post_pool(h, z, den, num, w1, b1, mg, mb, w2, b2, bf, cw, cb):
    nc = cw.shape[1]
    return pl.pallas_call(
        _post_pool_body,
        grid=(GRID,),
        in_specs=[_rows(F)] * 4 + [
            _full((F, 2 * F)), _full((8, 2 * F)), _full((8, 2 * F)),
            _full((8, 2 * F)), _full((2 * F, F)), _full((8, F)),
            _rows(F), _full((F, nc)), _full((8, nc))],
        out_specs=pl.BlockSpec((NUM_GRAPHS, nc), lambda i: (0, 0)),
        out_shape=jax.ShapeDtypeStruct((NUM_GRAPHS, nc), jnp.float32),
        scratch_shapes=[pltpu.VMEM((NUM_GRAPHS, F), jnp.float32)],
    )(h, z, den, num, w1, b1, mg, mb, w2, b2, bf, cw, cb)


def _bcast8(v):
    return jnp.broadcast_to(v.reshape(1, -1), (8, v.shape[-1]))


def kernel(x, edge_index, batch, params):
    src = jnp.concatenate([edge_index[0].astype(jnp.int32),
                           jnp.zeros((E_PAD,), jnp.int32)])
    dst = jnp.concatenate([edge_index[1].astype(jnp.int32),
                           jnp.full((E_PAD,), N_NODES, jnp.int32)])
    spread = jnp.arange(E_ROWS * EG, dtype=jnp.int32) & 63
    dst_a = jnp.where(dst < S, dst, TRASH_A + spread)
    dst_b = jnp.where(dst >= S, dst - S, TRASH_B + spread)
    srcp = src.reshape(E_ROWS, EG)
    dst2 = jnp.stack([dst_a, dst_b]).reshape(2, E_ROWS, EG)
    bf = jnp.broadcast_to(batch.astype(jnp.float32)[:, None], (N_NODES, F))

    layers = params['layers']
    l0 = layers[0]
    h, z, p, q = _enc_pre(
        x, params['enc_w'], _bcast8(params['enc_b']),
        _bcast8(l0['ln_g']), _bcast8(l0['ln_b']),
        jnp.full((8, F), l0['t'], jnp.float32))
    for i, lp in enumerate(layers):
        den, num = _edge_pass(p, q, srcp, dst2)
        den = den[:N_NODES]
        num = num[:N_NODES]
        if i + 1 < len(layers):
            ln = layers[i + 1]
            h, z, p, q = _post_pre(
                h, z, den, num, lp['w1'], _bcast8(lp['b1']),
                _bcast8(lp['mlp_ln_g']), _bcast8(lp['mlp_ln_b']),
                lp['w2'], _bcast8(lp['b2']),
                _bcast8(ln['ln_g']), _bcast8(ln['ln_b']),
                jnp.full((8, F), ln['t'], jnp.float32))
        else:
            out = _post_pool(
                h, z, den, num, lp['w1'], _bcast8(lp['b1']),
                _bcast8(lp['mlp_ln_g']), _bcast8(lp['mlp_ln_b']),
                lp['w2'], _bcast8(lp['b2']),
                bf, params['cls_w'], _bcast8(params['cls_b']))
    return out


# speedup vs baseline: 3.9116x; 1.3881x over previous
import functools

import jax
import jax.numpy as jnp
from jax import lax
from jax.experimental import pallas as pl
from jax.experimental.pallas import tpu as pltpu
from jax.experimental.pallas import tpu_sc as plsc

N_NODES = 10000
N_EDGES = 320000
F = 128
NUM_GRAPHS = 16
MSG_EPS = 1e-7
LN_EPS = 1e-5

NC = 2
NS = 16
EG = 64

E_ROWS = 5120
E_PAD = E_ROWS * EG - N_EDGES
ROWS_PER_TILE = E_ROWS // NS
W = 5120
S = 4992
N_PAD = S + W
TRASH_A = S
TRASH_B = N_NODES - S

BLK = 2000
GRID = N_NODES // BLK


ACC_ROWS_PER_TILE = W // NS
IDXB = 160
NBUF = 3


def _edge_body(pq_hbm, src2_hbm, dst2_hbm, zeros_hbm, dn_hbm,
               src_v, dst_v, rows_v, sem_g, acc):
    c = lax.axis_index("c")
    s = lax.axis_index("s")
    a0 = s * ACC_ROWS_PER_TILE
    asl = pl.ds(a0, ACC_ROWS_PER_TILE)
    r0 = s * ROWS_PER_TILE

    def buf(j):
        off = pl.multiple_of(lax.rem(j, NBUF) * EG, EG)
        return rows_v.at[pl.ds(off, EG)]

    def one_pass(p, carry):
        pltpu.sync_copy(zeros_hbm, acc.at[asl])
        plsc.subcore_barrier()

        def one_blk(b, carry2):
            rb = r0 + b * IDXB
            pltpu.sync_copy(src2_hbm.at[c, pl.ds(rb, IDXB)], src_v)
            pltpu.sync_copy(dst2_hbm.at[p, pl.ds(rb, IDXB)], dst_v)

            def body(i, carry3):
                @pl.when(i < IDXB)
                def _():
                    pltpu.async_copy(pq_hbm.at[src_v.at[i]], buf(i), sem_g)

                @pl.when(i >= 2)
                def _():
                    j = i - 2
                    pltpu.make_async_copy(pq_hbm.at[src_v.at[j]], buf(j),
                                          sem_g).wait()
                    pltpu.sync_copy(buf(j), acc.at[dst_v.at[j]], add=True)

                return carry3

            lax.fori_loop(0, IDXB + 2, body, 0)
            return carry2

        lax.fori_loop(0, ROWS_PER_TILE // IDXB, one_blk, 0)
        plsc.subcore_barrier()
        pltpu.sync_copy(acc.at[asl],
                        dn_hbm.at[c, pl.ds(p * S + a0, ACC_ROWS_PER_TILE)])
        return carry

    lax.fori_loop(0, 2, one_pass, 0)


def _edge_pass(pq, src2, dst2):
    zeros_blk = jnp.zeros((ACC_ROWS_PER_TILE, F), jnp.float32)
    mesh = plsc.VectorSubcoreMesh(core_axis_name="c", subcore_axis_name="s",
                                  num_cores=NC, num_subcores=NS)
    f = pl.kernel(
        _edge_body,
        out_type=jax.ShapeDtypeStruct((NC, N_PAD, F), jnp.float32),
        mesh=mesh,
        scratch_types=[
            pltpu.VMEM((IDXB, EG), jnp.int32),
            pltpu.VMEM((IDXB, EG), jnp.int32),
            pltpu.VMEM((NBUF * EG, F), jnp.float32),
            pltpu.SemaphoreType.DMA,
            pltpu.VMEM_SHARED((W, F), jnp.float32),
        ],
    )
    return f(pq, src2, dst2, zeros_blk)


def _ln(v, g, b):
    mu = jnp.mean(v, axis=-1, keepdims=True)
    var = jnp.mean((v - mu) ** 2, axis=-1, keepdims=True)
    return (v - mu) * lax.rsqrt(var + LN_EPS) * g + b


def _pre(h, ln_g, ln_b, t):
    z = jax.nn.relu(_ln(h, ln_g, ln_b))
    g = z + MSG_EPS
    p = jnp.exp(t * g)
    return z, p, g * p


def _mlp_post(h, z, den, num, w1, b1, mg, mb, w2, b2):
    aggr = num / (den + 1e-16)
    out = aggr + z
    u = jnp.dot(out, w1, preferred_element_type=jnp.float32) + b1
    u = jax.nn.relu(_ln(u, mg, mb))
    v = jnp.dot(u, w2, preferred_element_type=jnp.float32) + b2
    return h + v


def _enc_pre_body(x_ref, w_ref, b_ref, g_ref, bb_ref, t_ref,
                  h_ref, z_ref, p_ref, q_ref):
    h = jnp.dot(x_ref[...], w_ref[...],
                preferred_element_type=jnp.float32) + b_ref[0:1, :]
    z, p, q = _pre(h, g_ref[0:1, :], bb_ref[0:1, :], t_ref[0:1, :])
    h_ref[...] = h
    z_ref[...] = z
    p_ref[...] = p
    q_ref[...] = q


def _post_pre_body(h_ref, z_ref, den_ref, num_ref,
                   w1_ref, b1_ref, mg_ref, mb_ref, w2_ref, b2_ref,
                   g2_ref, bb2_ref, t2_ref,
                   h2_ref, z2_ref, p_ref, q_ref):
    hn = _mlp_post(h_ref[...], z_ref[...], den_ref[...], num_ref[...],
                   w1_ref[...], b1_ref[0:1, :], mg_ref[0:1, :], mb_ref[0:1, :],
                   w2_ref[...], b2_ref[0:1, :])
    z2, p, q = _pre(hn, g2_ref[0:1, :], bb2_ref[0:1, :], t2_ref[0:1, :])
    h2_ref[...] = hn
    z2_ref[...] = z2
    p_ref[...] = p
    q_ref[...] = q


def _post_pool_body(h_ref, z_ref, den_ref, num_ref,
                    w1_ref, b1_ref, mg_ref, mb_ref, w2_ref, b2_ref,
                    bf_ref, cw_ref, cb_ref,
                    out_ref, acc):
    i = pl.program_id(0)
    hn = _mlp_post(h_ref[...], z_ref[...], den_ref[...], num_ref[...],
                   w1_ref[...], b1_ref[0:1, :], mg_ref[0:1, :], mb_ref[0:1, :],
                   w2_ref[...], b2_ref[0:1, :])
    bvals = bf_ref[:, 0:NUM_GRAPHS]
    gid = lax.broadcasted_iota(jnp.int32, (BLK, NUM_GRAPHS), 1)
    oh = jnp.where(bvals == gid.astype(jnp.float32), 1.0, 0.0)
    part = lax.dot_general(oh, hn, (((0,), (0,)), ((), ())),
                           preferred_element_type=jnp.float32)

    @pl.when(i == 0)
    def _():
        acc[...] = part

    @pl.when(i > 0)
    def _():
        acc[...] = acc[...] + part

    out_ref[...] = jnp.dot(acc[...], cw_ref[...],
                           preferred_element_type=jnp.float32) + cb_ref[0:1, :]


def _full(shape):
    return pl.BlockSpec(shape, lambda i: (0, 0))


def _rows(width):
    return pl.BlockSpec((BLK, width), lambda i: (i, 0))


_N_SHAPE = jax.ShapeDtypeStruct((N_NODES, F), jnp.float32)


def _enc_pre(x, w, b, g, bb, t):
    return pl.pallas_call(
        _enc_pre_body,
        grid=(GRID,),
        in_specs=[_rows(F), _full((F, F)), _full((8, F)), _full((8, F)),
                  _full((8, F)), _full((8, F))],
        out_specs=[_rows(F)] * 4,
        out_shape=[_N_SHAPE] * 4,
    )(x, w, b, g, bb, t)


def _post_pre(h, z, den, num, w1, b1, mg, mb, w2, b2, g2, bb2, t2):
    return pl.pallas_call(
        _post_pre_body,
        grid=(GRID,),
        in_specs=[_rows(F)] * 4 + [
            _full((F, 2 * F)), _full((8, 2 * F)), _full((8, 2 * F)),
            _full((8, 2 * F)), _full((2 * F, F)), _full((8, F)),
            _full((8, F)), _full((8, F)), _full((8, F))],
        out_specs=[_rows(F)] * 4,
        out_shape=[_N_SHAPE] * 4,
    )(h, z, den, num, w1, b1, mg, mb, w2, b2, g2, bb2, t2)


def _post_pool(h, z, den, num, w1, b1, mg, mb, w2, b2, bf, cw, cb):
    nc = cw.shape[1]
    return pl.pallas_call(
        _post_pool_body,
        grid=(GRID,),
        in_specs=[_rows(F)] * 4 + [
            _full((F, 2 * F)), _full((8, 2 * F)), _full((8, 2 * F)),
            _full((8, 2 * F)), _full((2 * F, F)), _full((8, F)),
            _rows(F), _full((F, nc)), _full((8, nc))],
        out_specs=pl.BlockSpec((NUM_GRAPHS, nc), lambda i: (0, 0)),
        out_shape=jax.ShapeDtypeStruct((NUM_GRAPHS, nc), jnp.float32),
        scratch_shapes=[pltpu.VMEM((NUM_GRAPHS, F), jnp.float32)],
    )(h, z, den, num, w1, b1, mg, mb, w2, b2, bf, cw, cb)


def _bcast8(v):
    return jnp.broadcast_to(v.reshape(1, -1), (8, v.shape[-1]))


def kernel(x, edge_index, batch, params):
    src = jnp.concatenate([edge_index[0].astype(jnp.int32),
                           jnp.zeros((E_PAD,), jnp.int32)])
    dst = jnp.concatenate([edge_index[1].astype(jnp.int32),
                           jnp.full((E_PAD,), N_NODES, jnp.int32)])
    spread = jnp.arange(E_ROWS * EG, dtype=jnp.int32) & 63
    dst_a = jnp.where(dst < S, dst, TRASH_A + spread)
    dst_b = jnp.where(dst >= S, dst - S, TRASH_B + spread)
    src2 = jnp.stack([src, src + N_NODES]).reshape(2, E_ROWS, EG)
    dst2 = jnp.stack([dst_a, dst_b]).reshape(2, E_ROWS, EG)
    bf = jnp.broadcast_to(batch.astype(jnp.float32)[:, None], (N_NODES, F))

    layers = params['layers']
    l0 = layers[0]
    h, z, p, q = _enc_pre(
        x, params['enc_w'], _bcast8(params['enc_b']),
        _bcast8(l0['ln_g']), _bcast8(l0['ln_b']),
        jnp.full((8, F), l0['t'], jnp.float32))
    for i, lp in enumerate(layers):
        dn = _edge_pass(jnp.concatenate([p, q]), src2, dst2)
        den = dn[0, :N_NODES]
        num = dn[1, :N_NODES]
        if i + 1 < len(layers):
            ln = layers[i + 1]
            h, z, p, q = _post_pre(
                h, z, den, num, lp['w1'], _bcast8(lp['b1']),
                _bcast8(lp['mlp_ln_g']), _bcast8(lp['mlp_ln_b']),
                lp['w2'], _bcast8(lp['b2']),
                _bcast8(ln['ln_g']), _bcast8(ln['ln_b']),
                jnp.full((8, F), ln['t'], jnp.float32))
        else:
            out = _post_pool(
                h, z, den, num, lp['w1'], _bcast8(lp['b1']),
                _bcast8(lp['mlp_ln_g']), _bcast8(lp['mlp_ln_b']),
                lp['w2'], _bcast8(lp['b2']),
                bf, params['cls_w'], _bcast8(params['cls_b']))
    return out
